# baseline (speedup 1.0000x reference)
import jax
import jax.numpy as jnp
from jax.experimental import pallas as pl
from jax.experimental.pallas import tpu as pltpu

HEAD_NUM = 12
DIM_QK = 64
DIM_V = 64
SPAN = 32
LQ = 2048
LKV = 2048
DIM = 768

BQ = 256
SQ = 256
WN = SQ + SPAN
NB = LQ // BQ
NSUB = BQ // SQ


def _fused_kernel(q_ref, k_ref, v_ref, wq_ref, wk_ref, wv_ref, wo_ref,
                  out_ref, kp_scr, vp_scr, wqb, wkb, wvb, wob):
    i = pl.program_id(0)
    bf = jnp.bfloat16

    @pl.when(i == 0)
    def _init():
        kp_scr[pl.ds(0, SPAN), :] = jnp.zeros((SPAN, DIM), bf)
        vp_scr[pl.ds(0, SPAN), :] = jnp.zeros((SPAN, DIM), bf)
        scale2 = (1.0 / (DIM_QK ** 0.5)) * 1.4426950408889634
        wqb[...] = (wq_ref[...] * scale2).astype(bf)
        wkb[...] = wk_ref[...].astype(bf)
        wvb[...] = wv_ref[...].astype(bf)
        wob[...] = wo_ref[...].astype(bf)

    b = jnp.maximum(i - 1, 0)
    kb = jnp.minimum(i, NB - 1)

    kp_scr[pl.ds(pl.multiple_of(SPAN + kb * BQ, SPAN), BQ), :] = jnp.dot(
        k_ref[0].astype(bf), wkb[...],
        preferred_element_type=jnp.float32).astype(bf)
    vp_scr[pl.ds(pl.multiple_of(SPAN + kb * BQ, SPAN), BQ), :] = jnp.dot(
        v_ref[0].astype(bf), wvb[...],
        preferred_element_type=jnp.float32).astype(bf)

    qp = jnp.dot(q_ref[0].astype(bf), wqb[...],
                 preferred_element_type=jnp.float32).astype(bf)

    rr = jax.lax.broadcasted_iota(jnp.int32, (SQ, WN), 0)
    cc = jax.lax.broadcasted_iota(jnp.int32, (SQ, WN), 1)
    band = jnp.logical_and(cc >= rr + 1, cc <= rr + SPAN)
    ones_col = jnp.ones((WN, 1), bf)

    sub_outs = []
    for j in range(NSUB):
        wstart = pl.multiple_of(b * BQ + j * SQ, SQ)
        kwin = kp_scr[pl.ds(wstart, WN), :]
        vwin = vp_scr[pl.ds(wstart, WN), :]
        sub_band = jnp.logical_and(band, wstart - SPAN + cc >= 0)
        bias = jnp.where(sub_band, 0.0, -jnp.inf).astype(bf)

        head_outs = []
        for h in range(HEAD_NUM):
            qh = qp[j * SQ:(j + 1) * SQ, h * DIM_QK:(h + 1) * DIM_QK]
            kh = kwin[:, h * DIM_QK:(h + 1) * DIM_QK]
            s = jax.lax.dot_general(
                qh, kh, (((1,), (1,)), ((), ())),
                preferred_element_type=jnp.float32).astype(bf) + bias
            m = jnp.max(s, axis=1, keepdims=True)
            p = jnp.exp2(s - m)
            vh = jnp.concatenate(
                [vwin[:, h * DIM_V:(h + 1) * DIM_V], ones_col], axis=1)
            pv = jnp.dot(p, vh, preferred_element_type=jnp.float32)
            head_outs.append(
                pv[:, :DIM_V] * jax.lax.reciprocal(pv[:, DIM_V:DIM_V + 1]))
        sub_outs.append(jnp.concatenate(head_outs, axis=1))
    o = jnp.concatenate(sub_outs, axis=0).astype(bf)
    out_ref[0] = jnp.dot(o, wob[...], preferred_element_type=jnp.float32)


@jax.jit
def kernel(q, k, v, Wq, Wk, Wv, Wout):
    batch = q.shape[0]
    prev = lambda: pl.BlockSpec((1, BQ, DIM),
                                lambda i: (0, jnp.maximum(i - 1, 0), 0))
    cur = lambda: pl.BlockSpec((1, BQ, DIM),
                               lambda i: (0, jnp.minimum(i, NB - 1), 0))
    wspec = lambda: pl.BlockSpec((DIM, DIM), lambda i: (0, 0))
    out = pl.pallas_call(
        _fused_kernel,
        grid=(NB + 1,),
        in_specs=[prev(), cur(), cur(), wspec(), wspec(), wspec(), wspec()],
        out_specs=prev(),
        out_shape=jax.ShapeDtypeStruct((batch, LQ, DIM), jnp.float32),
        scratch_shapes=[
            pltpu.VMEM((SPAN + LKV, DIM), jnp.bfloat16),
            pltpu.VMEM((SPAN + LKV, DIM), jnp.bfloat16),
            pltpu.VMEM((DIM, DIM), jnp.bfloat16),
            pltpu.VMEM((DIM, DIM), jnp.bfloat16),
            pltpu.VMEM((DIM, DIM), jnp.bfloat16),
            pltpu.VMEM((DIM, DIM), jnp.bfloat16),
        ],
    )(q, k, v, Wq, Wk, Wv, Wout)
    return out


# speedup vs baseline: 1.1205x; 1.1205x over previous
import jax
import jax.numpy as jnp
from jax.experimental import pallas as pl
from jax.experimental.pallas import tpu as pltpu

HEAD_NUM = 12
DIM_QK = 64
DIM_V = 64
SPAN = 32
LQ = 2048
LKV = 2048
DIM = 768

BQ = 256
SQ = 256
WN = SQ + SPAN
NB = LQ // BQ
NSUB = BQ // SQ


def _fused_kernel(q_ref, k_ref, v_ref, wq_ref, wk_ref, wv_ref, wo_ref,
                  out_ref, kp_scr, vp_scr, wqb, wkb, wvb, wob):
    i = pl.program_id(0)
    bf = jnp.bfloat16

    @pl.when(i == 0)
    def _init():
        kp_scr[pl.ds(0, SPAN), :] = jnp.zeros((SPAN, DIM), bf)
        vp_scr[pl.ds(0, SPAN), :] = jnp.zeros((SPAN, DIM), bf)
        scale2 = (1.0 / (DIM_QK ** 0.5)) * 1.4426950408889634
        wqb[...] = (wq_ref[...] * scale2).astype(bf)
        wkb[...] = wk_ref[...].astype(bf)
        wvb[...] = wv_ref[...].astype(bf)
        wob[...] = wo_ref[...].astype(bf)

    kp_scr[pl.ds(SPAN + i * BQ, BQ), :] = jnp.dot(
        k_ref[0].astype(bf), wkb[...],
        preferred_element_type=jnp.float32).astype(bf)
    vp_scr[pl.ds(SPAN + i * BQ, BQ), :] = jnp.dot(
        v_ref[0].astype(bf), wvb[...],
        preferred_element_type=jnp.float32).astype(bf)

    qp = jnp.dot(q_ref[0].astype(bf), wqb[...],
                 preferred_element_type=jnp.float32).astype(bf)

    rr = jax.lax.broadcasted_iota(jnp.int32, (SQ, WN), 0)
    cc = jax.lax.broadcasted_iota(jnp.int32, (SQ, WN), 1)
    band = jnp.logical_and(cc >= rr + 1, cc <= rr + SPAN)
    ones_col = jnp.ones((WN, 1), bf)

    sub_outs = []
    for j in range(NSUB):
        wstart = pl.multiple_of(i * BQ + j * SQ, SQ)
        kwin = kp_scr[pl.ds(wstart, WN), :]
        vwin = vp_scr[pl.ds(wstart, WN), :]
        sub_band = jnp.logical_and(band, wstart - SPAN + cc >= 0)
        bias = jnp.where(sub_band, 0.0, -jnp.inf).astype(bf)

        head_outs = []
        for h in range(HEAD_NUM):
            qh = qp[j * SQ:(j + 1) * SQ, h * DIM_QK:(h + 1) * DIM_QK]
            kh = kwin[:, h * DIM_QK:(h + 1) * DIM_QK]
            s = jax.lax.dot_general(
                qh, kh, (((1,), (1,)), ((), ())),
                preferred_element_type=jnp.float32).astype(bf) + bias
            m = jnp.max(s, axis=1, keepdims=True)
            p = jnp.exp2(s - m)
            vh = jnp.concatenate(
                [vwin[:, h * DIM_V:(h + 1) * DIM_V], ones_col], axis=1)
            pv = jnp.dot(p, vh, preferred_element_type=jnp.float32)
            head_outs.append(
                pv[:, :DIM_V] * jax.lax.reciprocal(pv[:, DIM_V:DIM_V + 1]))
        sub_outs.append(jnp.concatenate(head_outs, axis=1))
    o = jnp.concatenate(sub_outs, axis=0).astype(bf)
    out_ref[0] = jnp.dot(o, wob[...], preferred_element_type=jnp.float32)


@jax.jit
def kernel(q, k, v, Wq, Wk, Wv, Wout):
    batch = q.shape[0]
    blk = lambda: pl.BlockSpec((1, BQ, DIM), lambda i: (0, i, 0))
    wspec = lambda: pl.BlockSpec((DIM, DIM), lambda i: (0, 0))
    out = pl.pallas_call(
        _fused_kernel,
        grid=(NB,),
        in_specs=[blk(), blk(), blk(), wspec(), wspec(), wspec(), wspec()],
        out_specs=blk(),
        out_shape=jax.ShapeDtypeStruct((batch, LQ, DIM), jnp.float32),
        scratch_shapes=[
            pltpu.VMEM((SPAN + LKV, DIM), jnp.bfloat16),
            pltpu.VMEM((SPAN + LKV, DIM), jnp.bfloat16),
            pltpu.VMEM((DIM, DIM), jnp.bfloat16),
            pltpu.VMEM((DIM, DIM), jnp.bfloat16),
            pltpu.VMEM((DIM, DIM), jnp.bfloat16),
            pltpu.VMEM((DIM, DIM), jnp.bfloat16),
        ],
    )(q, k, v, Wq, Wk, Wv, Wout)
    return out
